# baseline (speedup 1.0000x reference)
import functools

import jax
import jax.numpy as jnp
from jax import lax
from jax.experimental import pallas as pl
from jax.experimental.pallas import tpu as pltpu
from jax.experimental.pallas import tpu_sc as plsc

BN, C, N, K, CN, CL = 8, 128, 8192, 32, 256, 8
NQ = BN * CN
P = BN * N
EPS = 1e-5

NB = 512
BQ = 64

_NW = 32
QPW = NQ // _NW
CH = 128
NCH = QPW * K // CH


def _phase_a_body(x_ref, xt_ref):
    xt_ref[...] = x_ref[0].T


def _phase_a(xsc):
    return pl.pallas_call(
        _phase_a_body,
        grid=(BN, N // NB),
        in_specs=[pl.BlockSpec((1, C, NB), lambda b, j: (b, 0, j))],
        out_specs=pl.BlockSpec((NB, C), lambda b, j: (b * (N // NB) + j, 0)),
        out_shape=jax.ShapeDtypeStruct((P, C), jnp.float32),
    )(xsc)


def _adj_body(idx_ref, adj_ref):
    off = idx_ref[0] + pl.program_id(0) * N
    adj_ref[...] = jnp.concatenate(
        [off, jnp.zeros((NB, C - K), jnp.int32)], axis=1)


def _adj_offset(idx):
    return pl.pallas_call(
        _adj_body,
        grid=(BN, N // NB),
        in_specs=[pl.BlockSpec((1, NB, K), lambda b, j: (b, j, 0))],
        out_specs=pl.BlockSpec((NB, C), lambda b, j: (b * (N // NB) + j, 0)),
        out_shape=jax.ShapeDtypeStruct((P, C), jnp.int32),
    )(idx)


def _momentum(cur_feat, pre, mom_w, mom_gamma, mom_beta):
    cur4 = cur_feat.reshape(BN, CN, C).transpose(0, 2, 1)[..., None]
    pre4 = pre.reshape(BN, CN, C).transpose(0, 2, 1)[..., None]
    cat_feature = jnp.concatenate((cur4[:, :, :, 0], pre4[:, :, :, 0]), axis=1)
    mm = jnp.einsum('oc,bcn->bon', mom_w[:, :, 0], cat_feature)
    m = jnp.mean(mm, axis=(0, 2), keepdims=True)
    v = jnp.var(mm, axis=(0, 2), keepdims=True)
    mm = (mm - m) / jnp.sqrt(v + EPS) * mom_gamma.reshape(1, 2, 1) \
        + mom_beta.reshape(1, 2, 1)
    att = jax.nn.softmax(mm, axis=1).reshape(BN, 1, CN, 2)
    cat4 = jnp.concatenate((cur4, pre4), axis=-1)
    pre_new4 = jnp.sum(cat4 * att, axis=-1, keepdims=True)
    return pre_new4[..., 0].transpose(0, 2, 1).reshape(NQ, C)


def _pass1_body(rows_ref, cur_ref, pre_ref, wc_ref,
                raw_ref, sd_ref, sn_ref, n1_ref):
    rows = rows_ref[...]
    cur = cur_ref[...]
    pre = pre_ref[...]
    pre_rep = jnp.broadcast_to(pre[:, None, :], (BQ, K, C)).reshape(BQ * K, C)
    cat = jnp.concatenate([rows, pre_rep], axis=1)
    raw_ref[...] = lax.dot_general(cat, wc_ref[...],
                                   (((1,), (0,)), ((), ())))
    curv = cur - pre
    n1_ref[...] = jnp.sqrt(jnp.sum(curv * curv, axis=1, keepdims=True))
    r3 = rows.reshape(BQ, K, C)
    neigh = r3 - cur.reshape(BQ, 1, C)
    sd_ref[...] = lax.dot_general(curv, neigh,
                                  (((1,), (2,)), ((0,), (0,))))
    sn_ref[...] = jnp.sum(neigh * neigh, axis=2)


def _pass1(pick_rows, cur_feat, pre_new, wcat):
    g = NQ // BQ
    qspec = pl.BlockSpec((BQ, C), lambda i: (i, 0))
    cspec = pl.BlockSpec((BQ, 1), lambda i: (i, 0))
    kspec = pl.BlockSpec((BQ, K), lambda i: (i, 0))
    return pl.pallas_call(
        _pass1_body,
        grid=(g,),
        in_specs=[pl.BlockSpec((BQ * K, C), lambda i: (i, 0)),
                  qspec, qspec,
                  pl.BlockSpec((2 * C, 1), lambda i: (0, 0))],
        out_specs=[pl.BlockSpec((BQ * K, 1), lambda i: (i, 0)),
                   kspec, kspec, cspec],
        out_shape=[
            jax.ShapeDtypeStruct((NQ * K, 1), jnp.float32),
            jax.ShapeDtypeStruct((NQ, K), jnp.float32),
            jax.ShapeDtypeStruct((NQ, K), jnp.float32),
            jax.ShapeDtypeStruct((NQ, 1), jnp.float32),
        ],
    )(pick_rows, cur_feat, pre_new, wcat)


def _pass2_body(step, raw_ref, sd_ref, sn_ref, n1_ref, pidx_ref,
                g_ref, b_ref, fc_ref, sq_ref):
    raw = raw_ref[...]
    m = jnp.mean(raw)
    v = jnp.mean((raw - m) ** 2)
    nl = (raw - m) / jnp.sqrt(v + EPS) * g_ref[0, 0] + b_ref[0, 0]
    if step > 0:
        norm2 = jnp.sqrt(sn_ref[...])
        div = jnp.maximum(n1_ref[...] * norm2, 1e-8)
        sc = nl * jnp.clip(1.0 + sd_ref[...] / div, 0.0, 1.0)
    else:
        sc = nl
    mx = jnp.max(sc, axis=1, keepdims=True)
    kio = lax.broadcasted_iota(jnp.int32, (NQ, K), 1)
    ksel = jnp.min(jnp.where(sc == mx, kio, K), axis=1, keepdims=True)
    pidx = pidx_ref[...][:, :K]
    fc_ref[...] = jnp.sum(jnp.where(kio == ksel, pidx, 0),
                          axis=1, keepdims=True)
    y = 1.0 / jnp.sum(jnp.exp(sc - mx), axis=1, keepdims=True)
    sq_ref[...] = (1.0 - y) + y


def _pass2(step, raw, sd, sn, n1, pick_idx, ag, ab):
    kspec = pl.BlockSpec((NQ, K), lambda i: (0, 0))
    cspec = pl.BlockSpec((NQ, 1), lambda i: (0, 0))
    sspec = pl.BlockSpec((1, 1), lambda i: (0, 0))
    pspec = pl.BlockSpec((NQ, C), lambda i: (0, 0))
    return pl.pallas_call(
        functools.partial(_pass2_body, step),
        grid=(1,),
        in_specs=[kspec, kspec, kspec, cspec, pspec, sspec, sspec],
        out_specs=[cspec, cspec],
        out_shape=[jax.ShapeDtypeStruct((NQ, 1), jnp.int32),
                   jax.ShapeDtypeStruct((NQ, 1), jnp.float32)],
    )(raw, sd, sn, n1, pick_idx, ag, ab)


def _wid():
    return lax.axis_index("s") * 2 + lax.axis_index("c")


@functools.lru_cache(maxsize=None)
def _build_sc_gather_full():
    mesh = plsc.VectorSubcoreMesh(core_axis_name="c", subcore_axis_name="s")

    @functools.partial(
        pl.kernel, mesh=mesh,
        out_type=[
            jax.ShapeDtypeStruct((NQ * K, C), jnp.float32),
            jax.ShapeDtypeStruct((NQ, C), jnp.int32),
            jax.ShapeDtypeStruct((NQ, C), jnp.float32),
        ],
        scratch_types=[
            pltpu.VMEM((QPW,), jnp.int32),
            pltpu.VMEM((QPW, C), jnp.int32),
            pltpu.VMEM((NCH, CH), jnp.int32),
            pltpu.VMEM((QPW, C), jnp.float32),
            pltpu.VMEM((CH, C), jnp.float32),
            pltpu.VMEM((CH, C), jnp.float32),
            pltpu.SemaphoreType.DMA,
            pltpu.SemaphoreType.DMA,
        ],
    )
    def _sc_gather_full(x_hbm, adj_hbm, fc_hbm, pick_rows, pick_idx, cur_rows,
                        fc_v, idxr_v, idxf_v, curbuf, rb0, rb1, sem0, sem1):
        w = _wid()
        qb = w * QPW
        pltpu.sync_copy(fc_hbm.at[pl.ds(qb, QPW)], fc_v)
        pltpu.async_copy(adj_hbm.at[fc_v], idxr_v, sem0).wait()
        pltpu.sync_copy(idxr_v, pick_idx.at[pl.ds(qb, QPW)])
        pltpu.async_copy(x_hbm.at[fc_v], curbuf, sem0).wait()
        pltpu.sync_copy(curbuf, cur_rows.at[pl.ds(qb, QPW)])
        rows_per_chunk = CH // K
        for c in range(NCH):
            for j in range(rows_per_chunk):
                i = c * rows_per_chunk + j
                idxf_v[c, pl.ds(j * K, 16)] = idxr_v[i, pl.ds(0, 16)]
                idxf_v[c, pl.ds(j * K + 16, 16)] = idxr_v[i, pl.ds(16, 16)]
        bufs = (rb0, rb1)
        sems = (sem0, sem1)
        cps = [None, None]
        cps[0] = pltpu.async_copy(x_hbm.at[idxf_v.at[0]], rb0, sem0)
        for c in range(NCH):
            if c + 1 < NCH:
                cps[(c + 1) % 2] = pltpu.async_copy(
                    x_hbm.at[idxf_v.at[c + 1]], bufs[(c + 1) % 2],
                    sems[(c + 1) % 2])
            cps[c % 2].wait()
            pltpu.sync_copy(bufs[c % 2],
                            pick_rows.at[pl.ds(qb * K + c * CH, CH)])

    return _sc_gather_full


@functools.lru_cache(maxsize=None)
def _build_sc_gather_cur():
    mesh = plsc.VectorSubcoreMesh(core_axis_name="c", subcore_axis_name="s")

    @functools.partial(
        pl.kernel, mesh=mesh,
        out_type=jax.ShapeDtypeStruct((NQ, C), jnp.float32),
        scratch_types=[
            pltpu.VMEM((QPW,), jnp.int32),
            pltpu.VMEM((QPW, C), jnp.float32),
            pltpu.SemaphoreType.DMA,
        ],
    )
    def _sc_gather_cur(x_hbm, fc_hbm, cur_rows, fc_v, curbuf, sem):
        w = _wid()
        qb = w * QPW
        pltpu.sync_copy(fc_hbm.at[pl.ds(qb, QPW)], fc_v)
        pltpu.async_copy(x_hbm.at[fc_v], curbuf, sem).wait()
        pltpu.sync_copy(curbuf, cur_rows.at[pl.ds(qb, QPW)])

    return _sc_gather_cur


def _gather_step(Xt, adj, fc):
    return _build_sc_gather_full()(Xt, adj, fc)


def _gather_rows(Xt, fc):
    return _build_sc_gather_cur()(Xt, fc)


def kernel(x, xyz, idx, w_att, agent_w, agent_gamma, agent_beta,
           mom_w, mom_gamma, mom_beta):
    del xyz
    x_att = jax.nn.sigmoid(jnp.einsum('oc,bcn->bon', w_att[:, :, 0], x))
    xsc = x * x_att
    _, start_index = jax.lax.top_k(x_att[:, 0, :], CN)
    Xt = _phase_a(xsc)
    adj = _adj_offset(idx)
    fc = (start_index + (jnp.arange(BN) * N)[:, None]).reshape(NQ)
    wcat = agent_w[0, :, 0, 0].reshape(2 * C, 1)
    ag = agent_gamma.reshape(1, 1)
    ab = agent_beta.reshape(1, 1)

    outs = []
    pre = None
    sq = jnp.ones((NQ, 1), jnp.float32)
    for step in range(CL):
        pick_rows, pick_idx, cur_rows = _gather_step(Xt, adj, fc)
        cur_feat = cur_rows * sq
        if step == 0:
            pre_new = cur_feat
        else:
            outs.append(cur_feat)
            pre_new = _momentum(cur_feat, pre, mom_w, mom_gamma, mom_beta)
        raw, sd, sn, n1 = _pass1(pick_rows, cur_feat, pre_new, wcat)
        fc, sq = _pass2(step, raw.reshape(NQ, K), sd, sn, n1, pick_idx, ag, ab)
        fc = fc.reshape(NQ)
        pre = pre_new
    outs.append(_gather_rows(Xt, fc) * sq)
    res = jnp.stack(outs, axis=0)
    return res.reshape(CL, BN, CN, C).transpose(1, 3, 2, 0)


# speedup vs baseline: 1.0091x; 1.0091x over previous
import functools

import jax
import jax.numpy as jnp
from jax import lax
from jax.experimental import pallas as pl
from jax.experimental.pallas import tpu as pltpu
from jax.experimental.pallas import tpu_sc as plsc

BN, C, N, K, CN, CL = 8, 128, 8192, 32, 256, 8
NQ = BN * CN
P = BN * N
EPS = 1e-5

NB = 512
BQ = 64

_NW = 32
QPW = NQ // _NW
CH = 128
NCH = QPW * K // CH


def _phase_a_body(x_ref, xt_ref):
    xt_ref[...] = x_ref[0].T


def _phase_a(xsc):
    return pl.pallas_call(
        _phase_a_body,
        grid=(BN, N // NB),
        in_specs=[pl.BlockSpec((1, C, NB), lambda b, j: (b, 0, j))],
        out_specs=pl.BlockSpec((NB, C), lambda b, j: (b * (N // NB) + j, 0)),
        out_shape=jax.ShapeDtypeStruct((P, C), jnp.float32),
    )(xsc)


def _adj_body(idx_ref, adj_ref):
    off = idx_ref[0] + pl.program_id(0) * N
    adj_ref[...] = jnp.concatenate(
        [off, jnp.zeros((NB, C - K), jnp.int32)], axis=1)


def _adj_offset(idx):
    return pl.pallas_call(
        _adj_body,
        grid=(BN, N // NB),
        in_specs=[pl.BlockSpec((1, NB, K), lambda b, j: (b, j, 0))],
        out_specs=pl.BlockSpec((NB, C), lambda b, j: (b * (N // NB) + j, 0)),
        out_shape=jax.ShapeDtypeStruct((P, C), jnp.int32),
    )(idx)


def _momentum(cur_feat, pre, mom_w, mom_gamma, mom_beta):
    cur4 = cur_feat.reshape(BN, CN, C).transpose(0, 2, 1)[..., None]
    pre4 = pre.reshape(BN, CN, C).transpose(0, 2, 1)[..., None]
    cat_feature = jnp.concatenate((cur4[:, :, :, 0], pre4[:, :, :, 0]), axis=1)
    mm = jnp.einsum('oc,bcn->bon', mom_w[:, :, 0], cat_feature)
    m = jnp.mean(mm, axis=(0, 2), keepdims=True)
    v = jnp.var(mm, axis=(0, 2), keepdims=True)
    mm = (mm - m) / jnp.sqrt(v + EPS) * mom_gamma.reshape(1, 2, 1) \
        + mom_beta.reshape(1, 2, 1)
    att = jax.nn.softmax(mm, axis=1).reshape(BN, 1, CN, 2)
    cat4 = jnp.concatenate((cur4, pre4), axis=-1)
    pre_new4 = jnp.sum(cat4 * att, axis=-1, keepdims=True)
    return pre_new4[..., 0].transpose(0, 2, 1).reshape(NQ, C)


def _pass1_body(rows_ref, cur_ref, pre_ref, wc_ref,
                raw_ref, sd_ref, sn_ref, n1_ref):
    rows = rows_ref[...]
    cur = cur_ref[...]
    pre = pre_ref[...]
    pre_rep = jnp.broadcast_to(pre[:, None, :], (BQ, K, C)).reshape(BQ * K, C)
    cat = jnp.concatenate([rows, pre_rep], axis=1)
    raw_ref[...] = lax.dot_general(cat, wc_ref[...],
                                   (((1,), (0,)), ((), ())))
    curv = cur - pre
    n1_ref[...] = jnp.sqrt(jnp.sum(curv * curv, axis=1, keepdims=True))
    r3 = rows.reshape(BQ, K, C)
    neigh = r3 - cur.reshape(BQ, 1, C)
    sd_ref[...] = lax.dot_general(curv, neigh,
                                  (((1,), (2,)), ((0,), (0,))))
    sn_ref[...] = jnp.sum(neigh * neigh, axis=2)


def _pass1(pick_rows, cur_feat, pre_new, wcat):
    g = NQ // BQ
    qspec = pl.BlockSpec((BQ, C), lambda i: (i, 0))
    cspec = pl.BlockSpec((BQ, 1), lambda i: (i, 0))
    kspec = pl.BlockSpec((BQ, K), lambda i: (i, 0))
    return pl.pallas_call(
        _pass1_body,
        grid=(g,),
        in_specs=[pl.BlockSpec((BQ * K, C), lambda i: (i, 0)),
                  qspec, qspec,
                  pl.BlockSpec((2 * C, 1), lambda i: (0, 0))],
        out_specs=[pl.BlockSpec((BQ * K, 1), lambda i: (i, 0)),
                   kspec, kspec, cspec],
        out_shape=[
            jax.ShapeDtypeStruct((NQ * K, 1), jnp.float32),
            jax.ShapeDtypeStruct((NQ, K), jnp.float32),
            jax.ShapeDtypeStruct((NQ, K), jnp.float32),
            jax.ShapeDtypeStruct((NQ, 1), jnp.float32),
        ],
    )(pick_rows, cur_feat, pre_new, wcat)


def _pass2_body(step, raw_ref, sd_ref, sn_ref, n1_ref, pidx_ref,
                g_ref, b_ref, fc_ref, sq_ref):
    raw = raw_ref[...]
    m = jnp.mean(raw)
    v = jnp.mean((raw - m) ** 2)
    nl = (raw - m) / jnp.sqrt(v + EPS) * g_ref[0, 0] + b_ref[0, 0]
    if step > 0:
        norm2 = jnp.sqrt(sn_ref[...])
        div = jnp.maximum(n1_ref[...] * norm2, 1e-8)
        sc = nl * jnp.clip(1.0 + sd_ref[...] / div, 0.0, 1.0)
    else:
        sc = nl
    mx = jnp.max(sc, axis=1, keepdims=True)
    kio = lax.broadcasted_iota(jnp.int32, (NQ, K), 1)
    ksel = jnp.min(jnp.where(sc == mx, kio, K), axis=1, keepdims=True)
    pidx = pidx_ref[...][:, :K]
    fc_ref[...] = jnp.sum(jnp.where(kio == ksel, pidx, 0),
                          axis=1, keepdims=True)
    y = 1.0 / jnp.sum(jnp.exp(sc - mx), axis=1, keepdims=True)
    sq_ref[...] = (1.0 - y) + y


def _pass2(step, raw, sd, sn, n1, pick_idx, ag, ab):
    kspec = pl.BlockSpec((NQ, K), lambda i: (0, 0))
    cspec = pl.BlockSpec((NQ, 1), lambda i: (0, 0))
    sspec = pl.BlockSpec((1, 1), lambda i: (0, 0))
    pspec = pl.BlockSpec((NQ, C), lambda i: (0, 0))
    return pl.pallas_call(
        functools.partial(_pass2_body, step),
        grid=(1,),
        in_specs=[kspec, kspec, kspec, cspec, pspec, sspec, sspec],
        out_specs=[cspec, cspec],
        out_shape=[jax.ShapeDtypeStruct((NQ, 1), jnp.int32),
                   jax.ShapeDtypeStruct((NQ, 1), jnp.float32)],
    )(raw, sd, sn, n1, pick_idx, ag, ab)


def _wid():
    return lax.axis_index("s") * 2 + lax.axis_index("c")


@functools.lru_cache(maxsize=None)
def _build_sc_gather_full():
    mesh = plsc.VectorSubcoreMesh(core_axis_name="c", subcore_axis_name="s")

    @functools.partial(
        pl.kernel, mesh=mesh,
        out_type=[
            jax.ShapeDtypeStruct((NQ * K, C), jnp.float32),
            jax.ShapeDtypeStruct((NQ, C), jnp.int32),
            jax.ShapeDtypeStruct((NQ, C), jnp.float32),
        ],
        scratch_types=[
            pltpu.VMEM((QPW,), jnp.int32),
            pltpu.VMEM((QPW, C), jnp.int32),
            pltpu.VMEM((NCH, CH), jnp.int32),
            pltpu.VMEM((QPW, C), jnp.float32),
            pltpu.VMEM((CH, C), jnp.float32),
            pltpu.VMEM((CH, C), jnp.float32),
            pltpu.VMEM((CH, C), jnp.float32),
            pltpu.VMEM((CH, C), jnp.float32),
            pltpu.SemaphoreType.DMA,
            pltpu.SemaphoreType.DMA,
            pltpu.SemaphoreType.DMA,
            pltpu.SemaphoreType.DMA,
            pltpu.SemaphoreType.DMA,
            pltpu.SemaphoreType.DMA,
            pltpu.SemaphoreType.DMA,
            pltpu.SemaphoreType.DMA,
            pltpu.SemaphoreType.DMA,
        ],
    )
    def _sc_gather_full(x_hbm, adj_hbm, fc_hbm, pick_rows, pick_idx, cur_rows,
                        fc_v, idxr_v, idxf_v, curbuf, rb0, rb1, rb2, rb3,
                        g0, g1, g2, g3, s0, s1, s2, s3, csem):
        w = _wid()
        qb = w * QPW
        pltpu.sync_copy(fc_hbm.at[pl.ds(qb, QPW)], fc_v)
        pltpu.async_copy(adj_hbm.at[fc_v], idxr_v, csem).wait()
        rows_per_chunk = CH // K
        for c in range(NCH):
            for j in range(rows_per_chunk):
                i = c * rows_per_chunk + j
                idxf_v[c, pl.ds(j * K, 16)] = idxr_v[i, pl.ds(0, 16)]
                idxf_v[c, pl.ds(j * K + 16, 16)] = idxr_v[i, pl.ds(16, 16)]
        bufs = (rb0, rb1, rb2, rb3)
        gsems = (g0, g1, g2, g3)
        ssems = (s0, s1, s2, s3)
        nbuf = 4
        g = [None] * nbuf
        st = [None] * nbuf

        def drain(cc):
            b = cc % nbuf
            g[b].wait()
            st[b] = pltpu.async_copy(
                bufs[b], pick_rows.at[pl.ds(qb * K + cc * CH, CH)], ssems[b])

        for c in range(NCH):
            b = c % nbuf
            if st[b] is not None:
                st[b].wait()
            g[b] = pltpu.async_copy(x_hbm.at[idxf_v.at[c]], bufs[b], gsems[b])
            if c == 0:
                pltpu.sync_copy(idxr_v, pick_idx.at[pl.ds(qb, QPW)])
                pltpu.async_copy(x_hbm.at[fc_v], curbuf, csem).wait()
                pltpu.sync_copy(curbuf, cur_rows.at[pl.ds(qb, QPW)])
            if c >= nbuf - 1:
                drain(c - (nbuf - 1))
        for cc in range(NCH - nbuf + 1, NCH):
            drain(cc)
        for b in range(nbuf):
            if st[b] is not None:
                st[b].wait()

    return _sc_gather_full


@functools.lru_cache(maxsize=None)
def _build_sc_gather_cur():
    mesh = plsc.VectorSubcoreMesh(core_axis_name="c", subcore_axis_name="s")

    @functools.partial(
        pl.kernel, mesh=mesh,
        out_type=jax.ShapeDtypeStruct((NQ, C), jnp.float32),
        scratch_types=[
            pltpu.VMEM((QPW,), jnp.int32),
            pltpu.VMEM((QPW, C), jnp.float32),
            pltpu.SemaphoreType.DMA,
        ],
    )
    def _sc_gather_cur(x_hbm, fc_hbm, cur_rows, fc_v, curbuf, sem):
        w = _wid()
        qb = w * QPW
        pltpu.sync_copy(fc_hbm.at[pl.ds(qb, QPW)], fc_v)
        pltpu.async_copy(x_hbm.at[fc_v], curbuf, sem).wait()
        pltpu.sync_copy(curbuf, cur_rows.at[pl.ds(qb, QPW)])

    return _sc_gather_cur


def _gather_step(Xt, adj, fc):
    return _build_sc_gather_full()(Xt, adj, fc)


def _gather_rows(Xt, fc):
    return _build_sc_gather_cur()(Xt, fc)


def kernel(x, xyz, idx, w_att, agent_w, agent_gamma, agent_beta,
           mom_w, mom_gamma, mom_beta):
    del xyz
    x_att = jax.nn.sigmoid(jnp.einsum('oc,bcn->bon', w_att[:, :, 0], x))
    xsc = x * x_att
    _, start_index = jax.lax.top_k(x_att[:, 0, :], CN)
    Xt = _phase_a(xsc)
    adj = _adj_offset(idx)
    fc = (start_index + (jnp.arange(BN) * N)[:, None]).reshape(NQ)
    wcat = agent_w[0, :, 0, 0].reshape(2 * C, 1)
    ag = agent_gamma.reshape(1, 1)
    ab = agent_beta.reshape(1, 1)

    outs = []
    pre = None
    sq = jnp.ones((NQ, 1), jnp.float32)
    for step in range(CL):
        pick_rows, pick_idx, cur_rows = _gather_step(Xt, adj, fc)
        cur_feat = cur_rows * sq
        if step == 0:
            pre_new = cur_feat
        else:
            outs.append(cur_feat)
            pre_new = _momentum(cur_feat, pre, mom_w, mom_gamma, mom_beta)
        raw, sd, sn, n1 = _pass1(pick_rows, cur_feat, pre_new, wcat)
        fc, sq = _pass2(step, raw.reshape(NQ, K), sd, sn, n1, pick_idx, ag, ab)
        fc = fc.reshape(NQ)
        pre = pre_new
    outs.append(_gather_rows(Xt, fc) * sq)
    res = jnp.stack(outs, axis=0)
    return res.reshape(CL, BN, CN, C).transpose(1, 3, 2, 0)


# speedup vs baseline: 1.0888x; 1.0790x over previous
import functools

import jax
import jax.numpy as jnp
from jax import lax
from jax.experimental import pallas as pl
from jax.experimental.pallas import tpu as pltpu
from jax.experimental.pallas import tpu_sc as plsc

BN, C, N, K, CN, CL = 8, 128, 8192, 32, 256, 8
NQ = BN * CN
P = BN * N
EPS = 1e-5

NB = 512
BQ = 128

_NW = 32
QPW = NQ // _NW
CH = 128
NCH = QPW * K // CH


def _phase_a_body(x_ref, xt_ref):
    xt_ref[...] = x_ref[0].T


def _phase_a(xsc):
    return pl.pallas_call(
        _phase_a_body,
        grid=(BN, N // NB),
        in_specs=[pl.BlockSpec((1, C, NB), lambda b, j: (b, 0, j))],
        out_specs=pl.BlockSpec((NB, C), lambda b, j: (b * (N // NB) + j, 0)),
        out_shape=jax.ShapeDtypeStruct((P, C), jnp.float32),
    )(xsc)


def _adj_body(idx_ref, adj_ref):
    off = idx_ref[0] + pl.program_id(0) * N
    adj_ref[...] = jnp.concatenate(
        [off, jnp.zeros((NB, C - K), jnp.int32)], axis=1)


def _adj_offset(idx):
    return pl.pallas_call(
        _adj_body,
        grid=(BN, N // NB),
        in_specs=[pl.BlockSpec((1, NB, K), lambda b, j: (b, j, 0))],
        out_specs=pl.BlockSpec((NB, C), lambda b, j: (b * (N // NB) + j, 0)),
        out_shape=jax.ShapeDtypeStruct((P, C), jnp.int32),
    )(idx)


def _momentum(cur_feat, pre, mom_w, mom_gamma, mom_beta):
    cur4 = cur_feat.reshape(BN, CN, C).transpose(0, 2, 1)[..., None]
    pre4 = pre.reshape(BN, CN, C).transpose(0, 2, 1)[..., None]
    cat_feature = jnp.concatenate((cur4[:, :, :, 0], pre4[:, :, :, 0]), axis=1)
    mm = jnp.einsum('oc,bcn->bon', mom_w[:, :, 0], cat_feature)
    m = jnp.mean(mm, axis=(0, 2), keepdims=True)
    v = jnp.var(mm, axis=(0, 2), keepdims=True)
    mm = (mm - m) / jnp.sqrt(v + EPS) * mom_gamma.reshape(1, 2, 1) \
        + mom_beta.reshape(1, 2, 1)
    att = jax.nn.softmax(mm, axis=1).reshape(BN, 1, CN, 2)
    cat4 = jnp.concatenate((cur4, pre4), axis=-1)
    pre_new4 = jnp.sum(cat4 * att, axis=-1, keepdims=True)
    return pre_new4[..., 0].transpose(0, 2, 1).reshape(NQ, C)


def _pass1_full_body(rows_ref, cur_ref, pre_ref, wc_ref,
                     raw_ref, sd_ref, sn_ref, n1_ref):
    rows = rows_ref[...]
    cur = cur_ref[...]
    pre = pre_ref[...]
    pre_rep = jnp.broadcast_to(pre[:, None, :], (BQ, K, C)).reshape(BQ * K, C)
    cat = jnp.concatenate([rows, pre_rep], axis=1)
    raw_ref[...] = lax.dot_general(cat, wc_ref[...],
                                   (((1,), (0,)), ((), ())))
    curv = cur - pre
    n1_ref[...] = jnp.sqrt(jnp.sum(curv * curv, axis=1, keepdims=True))
    r3 = rows.reshape(BQ, K, C)
    neigh = r3 - cur.reshape(BQ, 1, C)
    sd_ref[...] = lax.dot_general(curv, neigh,
                                  (((1,), (2,)), ((0,), (0,))))
    sn_ref[...] = jnp.sum(neigh * neigh, axis=2)


def _pass1_raw_body(rows_ref, pre_ref, wc_ref, raw_ref):
    rows = rows_ref[...]
    pre = pre_ref[...]
    pre_rep = jnp.broadcast_to(pre[:, None, :], (BQ, K, C)).reshape(BQ * K, C)
    cat = jnp.concatenate([rows, pre_rep], axis=1)
    raw_ref[...] = lax.dot_general(cat, wc_ref[...],
                                   (((1,), (0,)), ((), ())))


def _pass1(step, pick_rows, cur_feat, pre_new, wcat):
    g = NQ // BQ
    qspec = pl.BlockSpec((BQ, C), lambda i: (i, 0))
    cspec = pl.BlockSpec((BQ, 1), lambda i: (i, 0))
    kspec = pl.BlockSpec((BQ, K), lambda i: (i, 0))
    rspec = pl.BlockSpec((BQ * K, C), lambda i: (i, 0))
    rawspec = pl.BlockSpec((BQ * K, 1), lambda i: (i, 0))
    wspec = pl.BlockSpec((2 * C, 1), lambda i: (0, 0))
    if step == 0:
        raw = pl.pallas_call(
            _pass1_raw_body,
            grid=(g,),
            in_specs=[rspec, qspec, wspec],
            out_specs=rawspec,
            out_shape=jax.ShapeDtypeStruct((NQ * K, 1), jnp.float32),
        )(pick_rows, pre_new, wcat)
        return raw, None, None, None
    return pl.pallas_call(
        _pass1_full_body,
        grid=(g,),
        in_specs=[rspec, qspec, qspec, wspec],
        out_specs=[rawspec, kspec, kspec, cspec],
        out_shape=[
            jax.ShapeDtypeStruct((NQ * K, 1), jnp.float32),
            jax.ShapeDtypeStruct((NQ, K), jnp.float32),
            jax.ShapeDtypeStruct((NQ, K), jnp.float32),
            jax.ShapeDtypeStruct((NQ, 1), jnp.float32),
        ],
    )(pick_rows, cur_feat, pre_new, wcat)


def _select_store(sc, pidx_ref, g_ref, b_ref, fc_ref, sq_ref):
    mx = jnp.max(sc, axis=1, keepdims=True)
    kio = lax.broadcasted_iota(jnp.int32, (NQ, K), 1)
    ksel = jnp.min(jnp.where(sc == mx, kio, K), axis=1, keepdims=True)
    pidx = pidx_ref[...][:, :K]
    fc_ref[...] = jnp.sum(jnp.where(kio == ksel, pidx, 0),
                          axis=1, keepdims=True)
    y = 1.0 / jnp.sum(jnp.exp(sc - mx), axis=1, keepdims=True)
    sq_ref[...] = (1.0 - y) + y


def _bn_logits(raw, g_ref, b_ref):
    m = jnp.mean(raw)
    v = jnp.mean((raw - m) ** 2)
    return (raw - m) / jnp.sqrt(v + EPS) * g_ref[0, 0] + b_ref[0, 0]


def _pass2_full_body(raw_ref, sd_ref, sn_ref, n1_ref, pidx_ref,
                     g_ref, b_ref, fc_ref, sq_ref):
    nl = _bn_logits(raw_ref[...], g_ref, b_ref)
    norm2 = jnp.sqrt(sn_ref[...])
    div = jnp.maximum(n1_ref[...] * norm2, 1e-8)
    sc = nl * jnp.clip(1.0 + sd_ref[...] / div, 0.0, 1.0)
    _select_store(sc, pidx_ref, g_ref, b_ref, fc_ref, sq_ref)


def _pass2_raw_body(raw_ref, pidx_ref, g_ref, b_ref, fc_ref, sq_ref):
    sc = _bn_logits(raw_ref[...], g_ref, b_ref)
    _select_store(sc, pidx_ref, g_ref, b_ref, fc_ref, sq_ref)


def _pass2(step, raw, sd, sn, n1, pick_idx, ag, ab):
    kspec = pl.BlockSpec((NQ, K), lambda i: (0, 0))
    cspec = pl.BlockSpec((NQ, 1), lambda i: (0, 0))
    sspec = pl.BlockSpec((1, 1), lambda i: (0, 0))
    pspec = pl.BlockSpec((NQ, C), lambda i: (0, 0))
    outs = dict(
        out_specs=[cspec, cspec],
        out_shape=[jax.ShapeDtypeStruct((NQ, 1), jnp.int32),
                   jax.ShapeDtypeStruct((NQ, 1), jnp.float32)],
    )
    if step == 0:
        return pl.pallas_call(
            _pass2_raw_body, grid=(1,),
            in_specs=[kspec, pspec, sspec, sspec], **outs,
        )(raw, pick_idx, ag, ab)
    return pl.pallas_call(
        _pass2_full_body, grid=(1,),
        in_specs=[kspec, kspec, kspec, cspec, pspec, sspec, sspec], **outs,
    )(raw, sd, sn, n1, pick_idx, ag, ab)


def _wid():
    return lax.axis_index("s") * 2 + lax.axis_index("c")


@functools.lru_cache(maxsize=None)
def _build_sc_gather_full():
    mesh = plsc.VectorSubcoreMesh(core_axis_name="c", subcore_axis_name="s")

    @functools.partial(
        pl.kernel, mesh=mesh,
        out_type=[
            jax.ShapeDtypeStruct((NQ * K, C), jnp.float32),
            jax.ShapeDtypeStruct((NQ, C), jnp.int32),
            jax.ShapeDtypeStruct((NQ, C), jnp.float32),
        ],
        scratch_types=[
            pltpu.VMEM((QPW,), jnp.int32),
            pltpu.VMEM((QPW, C), jnp.int32),
            pltpu.VMEM((NCH, CH), jnp.int32),
            pltpu.VMEM((QPW, C), jnp.float32),
            pltpu.VMEM((CH, C), jnp.float32),
            pltpu.VMEM((CH, C), jnp.float32),
            pltpu.VMEM((CH, C), jnp.float32),
            pltpu.VMEM((CH, C), jnp.float32),
            pltpu.SemaphoreType.DMA,
            pltpu.SemaphoreType.DMA,
            pltpu.SemaphoreType.DMA,
            pltpu.SemaphoreType.DMA,
            pltpu.SemaphoreType.DMA,
            pltpu.SemaphoreType.DMA,
            pltpu.SemaphoreType.DMA,
            pltpu.SemaphoreType.DMA,
            pltpu.SemaphoreType.DMA,
        ],
    )
    def _sc_gather_full(x_hbm, adj_hbm, fc_hbm, pick_rows, pick_idx, cur_rows,
                        fc_v, idxr_v, idxf_v, curbuf, rb0, rb1, rb2, rb3,
                        g0, g1, g2, g3, s0, s1, s2, s3, csem):
        w = _wid()
        qb = w * QPW
        pltpu.sync_copy(fc_hbm.at[pl.ds(qb, QPW)], fc_v)
        pltpu.async_copy(adj_hbm.at[fc_v], idxr_v, csem).wait()
        rows_per_chunk = CH // K
        for c in range(NCH):
            for j in range(rows_per_chunk):
                i = c * rows_per_chunk + j
                idxf_v[c, pl.ds(j * K, 16)] = idxr_v[i, pl.ds(0, 16)]
                idxf_v[c, pl.ds(j * K + 16, 16)] = idxr_v[i, pl.ds(16, 16)]
        bufs = (rb0, rb1, rb2, rb3)
        gsems = (g0, g1, g2, g3)
        ssems = (s0, s1, s2, s3)
        nbuf = 4
        g = [None] * nbuf
        st = [None] * nbuf

        def drain(cc):
            b = cc % nbuf
            g[b].wait()
            st[b] = pltpu.async_copy(
                bufs[b], pick_rows.at[pl.ds(qb * K + cc * CH, CH)], ssems[b])

        for c in range(NCH):
            b = c % nbuf
            if st[b] is not None:
                st[b].wait()
            g[b] = pltpu.async_copy(x_hbm.at[idxf_v.at[c]], bufs[b], gsems[b])
            if c == 0:
                pltpu.sync_copy(idxr_v, pick_idx.at[pl.ds(qb, QPW)])
                pltpu.async_copy(x_hbm.at[fc_v], curbuf, csem).wait()
                pltpu.sync_copy(curbuf, cur_rows.at[pl.ds(qb, QPW)])
            if c >= nbuf - 1:
                drain(c - (nbuf - 1))
        for cc in range(NCH - nbuf + 1, NCH):
            drain(cc)
        for b in range(nbuf):
            if st[b] is not None:
                st[b].wait()

    return _sc_gather_full


@functools.lru_cache(maxsize=None)
def _build_sc_gather_cur():
    mesh = plsc.VectorSubcoreMesh(core_axis_name="c", subcore_axis_name="s")

    @functools.partial(
        pl.kernel, mesh=mesh,
        out_type=jax.ShapeDtypeStruct((NQ, C), jnp.float32),
        scratch_types=[
            pltpu.VMEM((QPW,), jnp.int32),
            pltpu.VMEM((QPW, C), jnp.float32),
            pltpu.SemaphoreType.DMA,
        ],
    )
    def _sc_gather_cur(x_hbm, fc_hbm, cur_rows, fc_v, curbuf, sem):
        w = _wid()
        qb = w * QPW
        pltpu.sync_copy(fc_hbm.at[pl.ds(qb, QPW)], fc_v)
        pltpu.async_copy(x_hbm.at[fc_v], curbuf, sem).wait()
        pltpu.sync_copy(curbuf, cur_rows.at[pl.ds(qb, QPW)])

    return _sc_gather_cur


def _gather_step(Xt, adj, fc):
    return _build_sc_gather_full()(Xt, adj, fc)


def _gather_rows(Xt, fc):
    return _build_sc_gather_cur()(Xt, fc)


def kernel(x, xyz, idx, w_att, agent_w, agent_gamma, agent_beta,
           mom_w, mom_gamma, mom_beta):
    del xyz
    x_att = jax.nn.sigmoid(jnp.einsum('oc,bcn->bon', w_att[:, :, 0], x))
    xsc = x * x_att
    _, start_index = jax.lax.top_k(x_att[:, 0, :], CN)
    Xt = _phase_a(xsc)
    adj = _adj_offset(idx)
    fc = (start_index + (jnp.arange(BN) * N)[:, None]).reshape(NQ)
    wcat = agent_w[0, :, 0, 0].reshape(2 * C, 1)
    ag = agent_gamma.reshape(1, 1)
    ab = agent_beta.reshape(1, 1)

    outs = []
    pre = None
    sq = jnp.ones((NQ, 1), jnp.float32)
    for step in range(CL):
        pick_rows, pick_idx, cur_rows = _gather_step(Xt, adj, fc)
        cur_feat = cur_rows * sq
        if step == 0:
            pre_new = cur_feat
        else:
            outs.append(cur_feat)
            pre_new = _momentum(cur_feat, pre, mom_w, mom_gamma, mom_beta)
        raw, sd, sn, n1 = _pass1(step, pick_rows, cur_feat, pre_new, wcat)
        fc, sq = _pass2(step, raw.reshape(NQ, K), sd, sn, n1, pick_idx, ag, ab)
        fc = fc.reshape(NQ)
        pre = pre_new
    outs.append(_gather_rows(Xt, fc) * sq)
    res = jnp.stack(outs, axis=0)
    return res.reshape(CL, BN, CN, C).transpose(1, 3, 2, 0)


# speedup vs baseline: 1.0907x; 1.0018x over previous
import functools

import jax
import jax.numpy as jnp
from jax import lax
from jax.experimental import pallas as pl
from jax.experimental.pallas import tpu as pltpu
from jax.experimental.pallas import tpu_sc as plsc

BN, C, N, K, CN, CL = 8, 128, 8192, 32, 256, 8
NQ = BN * CN
P = BN * N
EPS = 1e-5

NB = 512
BQ = 128

_NW = 32
QPW = NQ // _NW
CH = 128
NCH = QPW * K // CH


def _phase_a_body(x_ref, xt_ref):
    xt_ref[...] = x_ref[0].T


def _phase_a(xsc):
    return pl.pallas_call(
        _phase_a_body,
        grid=(BN, N // NB),
        in_specs=[pl.BlockSpec((1, C, NB), lambda b, j: (b, 0, j))],
        out_specs=pl.BlockSpec((NB, C), lambda b, j: (b * (N // NB) + j, 0)),
        out_shape=jax.ShapeDtypeStruct((P, C), jnp.float32),
    )(xsc)


def _adj_body(idx_ref, adj_ref):
    off = idx_ref[0] + pl.program_id(0) * N
    adj_ref[...] = jnp.concatenate(
        [off, jnp.zeros((NB, C - K), jnp.int32)], axis=1)


def _adj_offset(idx):
    return pl.pallas_call(
        _adj_body,
        grid=(BN, N // NB),
        in_specs=[pl.BlockSpec((1, NB, K), lambda b, j: (b, j, 0))],
        out_specs=pl.BlockSpec((NB, C), lambda b, j: (b * (N // NB) + j, 0)),
        out_shape=jax.ShapeDtypeStruct((P, C), jnp.int32),
    )(idx)


def _momentum(cur_feat, pre, mom_w, mom_gamma, mom_beta):
    cur4 = cur_feat.reshape(BN, CN, C).transpose(0, 2, 1)[..., None]
    pre4 = pre.reshape(BN, CN, C).transpose(0, 2, 1)[..., None]
    cat_feature = jnp.concatenate((cur4[:, :, :, 0], pre4[:, :, :, 0]), axis=1)
    mm = jnp.einsum('oc,bcn->bon', mom_w[:, :, 0], cat_feature)
    m = jnp.mean(mm, axis=(0, 2), keepdims=True)
    v = jnp.var(mm, axis=(0, 2), keepdims=True)
    mm = (mm - m) / jnp.sqrt(v + EPS) * mom_gamma.reshape(1, 2, 1) \
        + mom_beta.reshape(1, 2, 1)
    att = jax.nn.softmax(mm, axis=1).reshape(BN, 1, CN, 2)
    cat4 = jnp.concatenate((cur4, pre4), axis=-1)
    pre_new4 = jnp.sum(cat4 * att, axis=-1, keepdims=True)
    return pre_new4[..., 0].transpose(0, 2, 1).reshape(NQ, C)


def _pass1_full_body(rows_ref, cur_ref, pre_ref, wc_ref,
                     raw_ref, sd_ref, sn_ref, n1_ref):
    rows = rows_ref[...]
    cur = cur_ref[...]
    pre = pre_ref[...]
    pre_rep = jnp.broadcast_to(pre[:, None, :], (BQ, K, C)).reshape(BQ * K, C)
    cat = jnp.concatenate([rows, pre_rep], axis=1)
    raw_ref[...] = lax.dot_general(cat, wc_ref[...],
                                   (((1,), (0,)), ((), ())))
    curv = cur - pre
    n1_ref[...] = jnp.sqrt(jnp.sum(curv * curv, axis=1, keepdims=True))
    r3 = rows.reshape(BQ, K, C)
    neigh = r3 - cur.reshape(BQ, 1, C)
    sd_ref[...] = lax.dot_general(curv, neigh,
                                  (((1,), (2,)), ((0,), (0,))))
    sn_ref[...] = jnp.sum(neigh * neigh, axis=2)


def _pass1_raw_body(rows_ref, pre_ref, wc_ref, raw_ref):
    rows = rows_ref[...]
    pre = pre_ref[...]
    pre_rep = jnp.broadcast_to(pre[:, None, :], (BQ, K, C)).reshape(BQ * K, C)
    cat = jnp.concatenate([rows, pre_rep], axis=1)
    raw_ref[...] = lax.dot_general(cat, wc_ref[...],
                                   (((1,), (0,)), ((), ())))


def _pass1(step, pick_rows, cur_feat, pre_new, wcat):
    g = NQ // BQ
    qspec = pl.BlockSpec((BQ, C), lambda i: (i, 0))
    cspec = pl.BlockSpec((BQ, 1), lambda i: (i, 0))
    kspec = pl.BlockSpec((BQ, K), lambda i: (i, 0))
    rspec = pl.BlockSpec((BQ * K, C), lambda i: (i, 0))
    rawspec = pl.BlockSpec((BQ * K, 1), lambda i: (i, 0))
    wspec = pl.BlockSpec((2 * C, 1), lambda i: (0, 0))
    if step == 0:
        raw = pl.pallas_call(
            _pass1_raw_body,
            grid=(g,),
            in_specs=[rspec, qspec, wspec],
            out_specs=rawspec,
            out_shape=jax.ShapeDtypeStruct((NQ * K, 1), jnp.float32),
        )(pick_rows, pre_new, wcat)
        return raw, None, None, None
    return pl.pallas_call(
        _pass1_full_body,
        grid=(g,),
        in_specs=[rspec, qspec, qspec, wspec],
        out_specs=[rawspec, kspec, kspec, cspec],
        out_shape=[
            jax.ShapeDtypeStruct((NQ * K, 1), jnp.float32),
            jax.ShapeDtypeStruct((NQ, K), jnp.float32),
            jax.ShapeDtypeStruct((NQ, K), jnp.float32),
            jax.ShapeDtypeStruct((NQ, 1), jnp.float32),
        ],
    )(pick_rows, cur_feat, pre_new, wcat)


def _select_store(sc, pidx_ref, g_ref, b_ref, fc_ref, sq_ref):
    mx = jnp.max(sc, axis=1, keepdims=True)
    kio = lax.broadcasted_iota(jnp.int32, (NQ, K), 1)
    ksel = jnp.min(jnp.where(sc == mx, kio, K), axis=1, keepdims=True)
    pidx = pidx_ref[...][:, :K]
    fc_ref[...] = jnp.sum(jnp.where(kio == ksel, pidx, 0),
                          axis=1, keepdims=True)
    y = 1.0 / jnp.sum(jnp.exp(sc - mx), axis=1, keepdims=True)
    sq_ref[...] = (1.0 - y) + y


def _bn_logits(raw, g_ref, b_ref):
    m = jnp.mean(raw)
    v = jnp.mean((raw - m) ** 2)
    return (raw - m) / jnp.sqrt(v + EPS) * g_ref[0, 0] + b_ref[0, 0]


def _pass2_full_body(raw_ref, sd_ref, sn_ref, n1_ref, pidx_ref,
                     g_ref, b_ref, fc_ref, sq_ref):
    nl = _bn_logits(raw_ref[...], g_ref, b_ref)
    norm2 = jnp.sqrt(sn_ref[...])
    div = jnp.maximum(n1_ref[...] * norm2, 1e-8)
    sc = nl * jnp.clip(1.0 + sd_ref[...] / div, 0.0, 1.0)
    _select_store(sc, pidx_ref, g_ref, b_ref, fc_ref, sq_ref)


def _pass2_raw_body(raw_ref, pidx_ref, g_ref, b_ref, fc_ref, sq_ref):
    sc = _bn_logits(raw_ref[...], g_ref, b_ref)
    _select_store(sc, pidx_ref, g_ref, b_ref, fc_ref, sq_ref)


def _pass2(step, raw, sd, sn, n1, pick_idx, ag, ab):
    kspec = pl.BlockSpec((NQ, K), lambda i: (0, 0))
    cspec = pl.BlockSpec((NQ, 1), lambda i: (0, 0))
    sspec = pl.BlockSpec((1, 1), lambda i: (0, 0))
    pspec = pl.BlockSpec((NQ, C), lambda i: (0, 0))
    outs = dict(
        out_specs=[cspec, cspec],
        out_shape=[jax.ShapeDtypeStruct((NQ, 1), jnp.int32),
                   jax.ShapeDtypeStruct((NQ, 1), jnp.float32)],
    )
    if step == 0:
        return pl.pallas_call(
            _pass2_raw_body, grid=(1,),
            in_specs=[kspec, pspec, sspec, sspec], **outs,
        )(raw, pick_idx, ag, ab)
    return pl.pallas_call(
        _pass2_full_body, grid=(1,),
        in_specs=[kspec, kspec, kspec, cspec, pspec, sspec, sspec], **outs,
    )(raw, sd, sn, n1, pick_idx, ag, ab)


def _wid():
    return lax.axis_index("s") * 2 + lax.axis_index("c")


@functools.lru_cache(maxsize=None)
def _build_sc_gather_full():
    mesh = plsc.VectorSubcoreMesh(core_axis_name="c", subcore_axis_name="s")

    @functools.partial(
        pl.kernel, mesh=mesh,
        out_type=[
            jax.ShapeDtypeStruct((NQ * K, C), jnp.float32),
            jax.ShapeDtypeStruct((NQ, C), jnp.int32),
            jax.ShapeDtypeStruct((NQ, C), jnp.float32),
        ],
        scratch_types=[
            pltpu.VMEM((QPW,), jnp.int32),
            pltpu.VMEM((QPW, C), jnp.int32),
            pltpu.VMEM((NCH, CH), jnp.int32),
            pltpu.VMEM((QPW, C), jnp.float32),
            pltpu.VMEM((CH, C), jnp.float32),
            pltpu.VMEM((CH, C), jnp.float32),
            pltpu.VMEM((CH, C), jnp.float32),
            pltpu.VMEM((CH, C), jnp.float32),
            pltpu.VMEM((CH, C), jnp.float32),
            pltpu.VMEM((CH, C), jnp.float32),
            pltpu.SemaphoreType.DMA,
            pltpu.SemaphoreType.DMA,
            pltpu.SemaphoreType.DMA,
            pltpu.SemaphoreType.DMA,
            pltpu.SemaphoreType.DMA,
            pltpu.SemaphoreType.DMA,
            pltpu.SemaphoreType.DMA,
            pltpu.SemaphoreType.DMA,
            pltpu.SemaphoreType.DMA,
            pltpu.SemaphoreType.DMA,
            pltpu.SemaphoreType.DMA,
            pltpu.SemaphoreType.DMA,
            pltpu.SemaphoreType.DMA,
        ],
    )
    def _sc_gather_full(x_hbm, adj_hbm, fc_hbm, pick_rows, pick_idx, cur_rows,
                        fc_v, idxr_v, idxf_v, curbuf,
                        rb0, rb1, rb2, rb3, rb4, rb5,
                        g0, g1, g2, g3, g4, g5,
                        s0, s1, s2, s3, s4, s5, csem):
        w = _wid()
        qb = w * QPW
        pltpu.sync_copy(fc_hbm.at[pl.ds(qb, QPW)], fc_v)
        pltpu.async_copy(adj_hbm.at[fc_v], idxr_v, csem).wait()
        rows_per_chunk = CH // K
        for c in range(NCH):
            for j in range(rows_per_chunk):
                i = c * rows_per_chunk + j
                idxf_v[c, pl.ds(j * K, 16)] = idxr_v[i, pl.ds(0, 16)]
                idxf_v[c, pl.ds(j * K + 16, 16)] = idxr_v[i, pl.ds(16, 16)]
        bufs = (rb0, rb1, rb2, rb3, rb4, rb5)
        gsems = (g0, g1, g2, g3, g4, g5)
        ssems = (s0, s1, s2, s3, s4, s5)
        nbuf = 6
        g = [None] * nbuf
        st = [None] * nbuf

        def drain(cc):
            b = cc % nbuf
            g[b].wait()
            st[b] = pltpu.async_copy(
                bufs[b], pick_rows.at[pl.ds(qb * K + cc * CH, CH)], ssems[b])

        for c in range(NCH):
            b = c % nbuf
            if st[b] is not None:
                st[b].wait()
            g[b] = pltpu.async_copy(x_hbm.at[idxf_v.at[c]], bufs[b], gsems[b])
            if c == 0:
                pltpu.sync_copy(idxr_v, pick_idx.at[pl.ds(qb, QPW)])
                pltpu.async_copy(x_hbm.at[fc_v], curbuf, csem).wait()
                pltpu.sync_copy(curbuf, cur_rows.at[pl.ds(qb, QPW)])
            if c >= nbuf - 1:
                drain(c - (nbuf - 1))
        for cc in range(NCH - nbuf + 1, NCH):
            drain(cc)
        for b in range(nbuf):
            if st[b] is not None:
                st[b].wait()

    return _sc_gather_full


@functools.lru_cache(maxsize=None)
def _build_sc_gather_cur():
    mesh = plsc.VectorSubcoreMesh(core_axis_name="c", subcore_axis_name="s")

    @functools.partial(
        pl.kernel, mesh=mesh,
        out_type=jax.ShapeDtypeStruct((NQ, C), jnp.float32),
        scratch_types=[
            pltpu.VMEM((QPW,), jnp.int32),
            pltpu.VMEM((QPW, C), jnp.float32),
            pltpu.SemaphoreType.DMA,
        ],
    )
    def _sc_gather_cur(x_hbm, fc_hbm, cur_rows, fc_v, curbuf, sem):
        w = _wid()
        qb = w * QPW
        pltpu.sync_copy(fc_hbm.at[pl.ds(qb, QPW)], fc_v)
        pltpu.async_copy(x_hbm.at[fc_v], curbuf, sem).wait()
        pltpu.sync_copy(curbuf, cur_rows.at[pl.ds(qb, QPW)])

    return _sc_gather_cur


def _gather_step(Xt, adj, fc):
    return _build_sc_gather_full()(Xt, adj, fc)


def _gather_rows(Xt, fc):
    return _build_sc_gather_cur()(Xt, fc)


def kernel(x, xyz, idx, w_att, agent_w, agent_gamma, agent_beta,
           mom_w, mom_gamma, mom_beta):
    del xyz
    x_att = jax.nn.sigmoid(jnp.einsum('oc,bcn->bon', w_att[:, :, 0], x))
    xsc = x * x_att
    _, start_index = jax.lax.top_k(x_att[:, 0, :], CN)
    Xt = _phase_a(xsc)
    adj = _adj_offset(idx)
    fc = (start_index + (jnp.arange(BN) * N)[:, None]).reshape(NQ)
    wcat = agent_w[0, :, 0, 0].reshape(2 * C, 1)
    ag = agent_gamma.reshape(1, 1)
    ab = agent_beta.reshape(1, 1)

    outs = []
    pre = None
    sq = jnp.ones((NQ, 1), jnp.float32)
    for step in range(CL):
        pick_rows, pick_idx, cur_rows = _gather_step(Xt, adj, fc)
        cur_feat = cur_rows * sq
        if step == 0:
            pre_new = cur_feat
        else:
            outs.append(cur_feat)
            pre_new = _momentum(cur_feat, pre, mom_w, mom_gamma, mom_beta)
        raw, sd, sn, n1 = _pass1(step, pick_rows, cur_feat, pre_new, wcat)
        fc, sq = _pass2(step, raw.reshape(NQ, K), sd, sn, n1, pick_idx, ag, ab)
        fc = fc.reshape(NQ)
        pre = pre_new
    outs.append(_gather_rows(Xt, fc) * sq)
    res = jnp.stack(outs, axis=0)
    return res.reshape(CL, BN, CN, C).transpose(1, 3, 2, 0)
